# baseline (speedup 1.0000x reference)
import functools

import jax
import jax.numpy as jnp
from jax import lax
from jax.experimental import pallas as pl
from jax.experimental.pallas import tpu as pltpu
from jax.experimental.pallas import tpu_sc as plsc

SEQ = 8192
D = 2048
LANES = 16
NC = 2
NS = 16
NW = NC * NS
ROWS_PER_W = SEQ // NW
CHUNK = 8
NSTEPS = ROWS_PER_W // CHUNK
NBUF = 3
NGROUPS = NSTEPS // NBUF
STRIPS = D // LANES


def _pe_body(x_hbm, w_hbm, pos_hbm, out_hbm, idx_v, *bufs):
    x_v = bufs[0:NBUF]
    g_v = bufs[NBUF:2 * NBUF]
    gs = bufs[2 * NBUF:3 * NBUF]
    xs = bufs[3 * NBUF:4 * NBUF]
    os_ = bufs[4 * NBUF:5 * NBUF]

    wid = lax.axis_index("s") * NC + lax.axis_index("c")
    base = wid * ROWS_PER_W

    pltpu.sync_copy(pos_hbm.at[pl.ds(base, ROWS_PER_W)], idx_v)

    def start_loads(s, b):
        row0 = base + s * CHUNK
        pltpu.async_copy(w_hbm.at[idx_v.at[pl.ds(s * CHUNK, CHUNK)]],
                         g_v[b], gs[b])
        pltpu.async_copy(x_hbm.at[pl.ds(row0, CHUNK)], x_v[b], xs[b])

    def wait_loads(s, b):
        pltpu.make_async_copy(w_hbm.at[idx_v.at[pl.ds(s * CHUNK, CHUNK)]],
                              g_v[b], gs[b]).wait()
        pltpu.make_async_copy(x_hbm.at[pl.ds(base, CHUNK)], x_v[b],
                              xs[b]).wait()

    def add_chunk(b):
        def row_body(r, c2):
            for c in range(STRIPS):
                sl = pl.ds(c * LANES, LANES)
                x_v[b][r, sl] = x_v[b][r, sl] + g_v[b][r, sl]
            return c2
        lax.fori_loop(0, CHUNK, row_body, 0, unroll=False)

    def start_store(s, b):
        row0 = base + s * CHUNK
        pltpu.async_copy(x_v[b], out_hbm.at[pl.ds(row0, CHUNK)], os_[b])

    def wait_store(b):
        pltpu.make_async_copy(x_v[b], out_hbm.at[pl.ds(base, CHUNK)],
                              os_[b]).wait()

    def step(s, b, refill):
        wait_loads(s, b)
        add_chunk(b)
        start_store(s, b)
        if refill:
            b2 = (b + 2) % NBUF

            @pl.when(s < NSTEPS - 2)
            def _():
                @pl.when(s > 0)
                def _():
                    wait_store(b2)
                start_loads(s + 2, b2)

    start_loads(0, 0)
    start_loads(1, 1)

    def group(p, carry):
        for j in range(NBUF):
            step(NBUF * p + j, j, refill=True)
        return carry

    lax.fori_loop(0, NGROUPS, group, 0)

    step(NSTEPS - 2, 0, refill=False)
    step(NSTEPS - 1, 1, refill=False)

    for b in range(NBUF):
        wait_store(b)


@jax.jit
def kernel(x, W, pos):
    mesh = plsc.VectorSubcoreMesh(core_axis_name="c", subcore_axis_name="s")
    f = pl.kernel(
        _pe_body,
        mesh=mesh,
        out_type=jax.ShapeDtypeStruct((SEQ, D), jnp.float32),
        scratch_types=(
            [pltpu.VMEM((ROWS_PER_W,), jnp.int32)]
            + [pltpu.VMEM((CHUNK, D), jnp.float32) for _ in range(2 * NBUF)]
            + [pltpu.SemaphoreType.DMA for _ in range(3 * NBUF)]
        ),
    )
    return f(x, W, pos)


# speedup vs baseline: 1.1803x; 1.1803x over previous
import functools

import jax
import jax.numpy as jnp
from jax import lax
from jax.experimental import pallas as pl
from jax.experimental.pallas import tpu as pltpu
from jax.experimental.pallas import tpu_sc as plsc

SEQ = 8192
D = 2048
LANES = 16
NC = 2
NS = 16
NW = NC * NS
ROWS_PER_W = SEQ // NW
CHUNK = 8
NSTEPS = ROWS_PER_W // CHUNK
NPAIRS = NSTEPS // 2
STRIPS = D // LANES


def _pe_body(x_hbm, w_hbm, pos_hbm, out_hbm,
             idx_v,
             x0, g0, o0, x1, g1, o1,
             gs0, xs0, os0, gs1, xs1, os1):
    wid = lax.axis_index("s") * NC + lax.axis_index("c")
    base = wid * ROWS_PER_W

    pltpu.sync_copy(pos_hbm.at[pl.ds(base, ROWS_PER_W)], idx_v)

    def start_loads(s, x_v, g_v, gsem, xsem):
        row0 = base + s * CHUNK
        pltpu.async_copy(w_hbm.at[idx_v.at[pl.ds(s * CHUNK, CHUNK)]], g_v, gsem)
        pltpu.async_copy(x_hbm.at[pl.ds(row0, CHUNK)], x_v, xsem)

    def wait_loads(s, x_v, g_v, gsem, xsem):
        pltpu.make_async_copy(w_hbm.at[idx_v.at[pl.ds(s * CHUNK, CHUNK)]],
                              g_v, gsem).wait()
        pltpu.make_async_copy(x_hbm.at[pl.ds(base, CHUNK)], x_v, xsem).wait()

    def add_chunk(x_v, g_v, o_v):
        def row_body(r2, c2):
            ra = 2 * r2
            rb = ra + 1
            for c in range(STRIPS):
                sl = pl.ds(c * LANES, LANES)
                o_v[ra, sl] = x_v[ra, sl] + g_v[ra, sl]
                o_v[rb, sl] = x_v[rb, sl] + g_v[rb, sl]
            return c2
        lax.fori_loop(0, CHUNK // 2, row_body, 0, unroll=False)

    def start_store(s, o_v, osem):
        row0 = base + s * CHUNK
        pltpu.async_copy(o_v, out_hbm.at[pl.ds(row0, CHUNK)], osem)

    def wait_store(o_v, osem):
        pltpu.make_async_copy(o_v, out_hbm.at[pl.ds(base, CHUNK)], osem).wait()

    start_loads(0, x0, g0, gs0, xs0)
    start_loads(1, x1, g1, gs1, xs1)

    def pair(p, carry):
        s0 = 2 * p
        s1 = s0 + 1

        wait_loads(s0, x0, g0, gs0, xs0)

        @pl.when(p > 0)
        def _():
            wait_store(o0, os0)

        add_chunk(x0, g0, o0)
        start_store(s0, o0, os0)

        @pl.when(p < NPAIRS - 1)
        def _():
            start_loads(s0 + 2, x0, g0, gs0, xs0)

        wait_loads(s1, x1, g1, gs1, xs1)

        @pl.when(p > 0)
        def _():
            wait_store(o1, os1)

        add_chunk(x1, g1, o1)
        start_store(s1, o1, os1)

        @pl.when(p < NPAIRS - 1)
        def _():
            start_loads(s1 + 2, x1, g1, gs1, xs1)

        return carry

    lax.fori_loop(0, NPAIRS, pair, 0)

    wait_store(o0, os0)
    wait_store(o1, os1)


@jax.jit
def kernel(x, W, pos):
    mesh = plsc.VectorSubcoreMesh(core_axis_name="c", subcore_axis_name="s")
    f = pl.kernel(
        _pe_body,
        mesh=mesh,
        out_type=jax.ShapeDtypeStruct((SEQ, D), jnp.float32),
        scratch_types=[
            pltpu.VMEM((ROWS_PER_W,), jnp.int32),
            pltpu.VMEM((CHUNK, D), jnp.float32),
            pltpu.VMEM((CHUNK, D), jnp.float32),
            pltpu.VMEM((CHUNK, D), jnp.float32),
            pltpu.VMEM((CHUNK, D), jnp.float32),
            pltpu.VMEM((CHUNK, D), jnp.float32),
            pltpu.VMEM((CHUNK, D), jnp.float32),
            pltpu.SemaphoreType.DMA,
            pltpu.SemaphoreType.DMA,
            pltpu.SemaphoreType.DMA,
            pltpu.SemaphoreType.DMA,
            pltpu.SemaphoreType.DMA,
            pltpu.SemaphoreType.DMA,
        ],
    )
    return f(x, W, pos)


# speedup vs baseline: 1.3938x; 1.1809x over previous
import functools

import jax
import jax.numpy as jnp
from jax import lax
from jax.experimental import pallas as pl
from jax.experimental.pallas import tpu as pltpu
from jax.experimental.pallas import tpu_sc as plsc

SEQ = 8192
D = 2048
LANES = 16
NC = 2
NS = 16
NW = NC * NS
ROWS_PER_W = SEQ // NW
CHUNK = 8
NSTEPS = ROWS_PER_W // CHUNK
NPAIRS = NSTEPS // 2
STRIPS = D // LANES


def _pe_body(x_hbm, w_hbm, pos_hbm, out_hbm,
             idx_v,
             x0, g0, o0, x1, g1, o1,
             gs0, xs0, os0, gs1, xs1, os1):
    wid = lax.axis_index("s") * NC + lax.axis_index("c")
    base = wid * ROWS_PER_W

    pltpu.sync_copy(pos_hbm.at[pl.ds(base, ROWS_PER_W)], idx_v)

    def start_loads(s, x_v, g_v, gsem, xsem):
        row0 = base + s * CHUNK
        pltpu.async_copy(w_hbm.at[idx_v.at[pl.ds(s * CHUNK, CHUNK)]], g_v, gsem)
        pltpu.async_copy(x_hbm.at[pl.ds(row0, CHUNK)], x_v, xsem)

    def wait_loads(s, x_v, g_v, gsem, xsem):
        pltpu.make_async_copy(w_hbm.at[idx_v.at[pl.ds(s * CHUNK, CHUNK)]],
                              g_v, gsem).wait()
        pltpu.make_async_copy(x_hbm.at[pl.ds(base, CHUNK)], x_v, xsem).wait()

    def add_chunk(x_v, g_v, o_v):
        def row_body(r, c2):
            for c in range(STRIPS):
                sl = pl.ds(c * LANES, LANES)
                o_v[r, sl] = x_v[r, sl] + g_v[r, sl]
            return c2
        lax.fori_loop(0, CHUNK, row_body, 0, unroll=False)

    def start_store(s, o_v, osem):
        row0 = base + s * CHUNK
        pltpu.async_copy(o_v, out_hbm.at[pl.ds(row0, CHUNK)], osem)

    def wait_store(o_v, osem):
        pltpu.make_async_copy(o_v, out_hbm.at[pl.ds(base, CHUNK)], osem).wait()

    start_loads(0, x0, g0, gs0, xs0)
    start_loads(1, x1, g1, gs1, xs1)

    def pair(p, carry):
        s0 = 2 * p
        s1 = s0 + 1

        wait_loads(s0, x0, g0, gs0, xs0)

        @pl.when(p > 0)
        def _():
            wait_store(o0, os0)

        add_chunk(x0, g0, o0)
        start_store(s0, o0, os0)

        @pl.when(p < NPAIRS - 1)
        def _():
            start_loads(s0 + 2, x0, g0, gs0, xs0)

        wait_loads(s1, x1, g1, gs1, xs1)

        @pl.when(p > 0)
        def _():
            wait_store(o1, os1)

        add_chunk(x1, g1, o1)
        start_store(s1, o1, os1)

        @pl.when(p < NPAIRS - 1)
        def _():
            start_loads(s1 + 2, x1, g1, gs1, xs1)

        return carry

    lax.fori_loop(0, NPAIRS, pair, 0)

    wait_store(o0, os0)
    wait_store(o1, os1)


@jax.jit
def kernel(x, W, pos):
    mesh = plsc.VectorSubcoreMesh(core_axis_name="c", subcore_axis_name="s")
    f = pl.kernel(
        _pe_body,
        mesh=mesh,
        out_type=jax.ShapeDtypeStruct((SEQ, D), jnp.float32),
        scratch_types=[
            pltpu.VMEM((ROWS_PER_W,), jnp.int32),
            pltpu.VMEM((CHUNK, D), jnp.float32),
            pltpu.VMEM((CHUNK, D), jnp.float32),
            pltpu.VMEM((CHUNK, D), jnp.float32),
            pltpu.VMEM((CHUNK, D), jnp.float32),
            pltpu.VMEM((CHUNK, D), jnp.float32),
            pltpu.VMEM((CHUNK, D), jnp.float32),
            pltpu.SemaphoreType.DMA,
            pltpu.SemaphoreType.DMA,
            pltpu.SemaphoreType.DMA,
            pltpu.SemaphoreType.DMA,
            pltpu.SemaphoreType.DMA,
            pltpu.SemaphoreType.DMA,
        ],
    )
    return f(x, W, pos)
